# baseline (speedup 1.0000x reference)
import functools

import jax
import jax.numpy as jnp
from jax import lax
from jax.experimental import pallas as pl
from jax.experimental.pallas import tpu as pltpu
from jax.experimental.pallas import tpu_sc as plsc

VOCAB_ROWS = 100002
DIM = 64
B = 16384
L = 50

NUM_CORES = 2
NUM_SUBCORES = 16
NW = NUM_CORES * NUM_SUBCORES
WIN = B // NW
CBLK = 256
NH = WIN // CBLK
NCHUNK = L * NH
NGRP = CBLK // 16


def _make_kernel():
  mesh = plsc.VectorSubcoreMesh(core_axis_name="c", subcore_axis_name="s")

  @functools.partial(
      pl.kernel,
      mesh=mesh,
      compiler_params=pltpu.CompilerParams(use_tc_tiling_on_sc=False,
                                           needs_layout_passes=False),
      out_type=jax.ShapeDtypeStruct((L, DIM // 8, B // 128, 8, 128),
                                    jnp.float32),
      scratch_types=[
          pltpu.VMEM((L, WIN), jnp.int32),
          pltpu.VMEM((CBLK, DIM), jnp.float32),
          pltpu.VMEM((CBLK, DIM), jnp.float32),
          pltpu.VMEM((DIM // 8, CBLK // 128, 8, 128), jnp.float32),
          pltpu.VMEM((DIM // 8, CBLK // 128, 8, 128), jnp.float32),
          pltpu.SemaphoreType.DMA,
          pltpu.SemaphoreType.DMA,
          pltpu.SemaphoreType.DMA,
          pltpu.SemaphoreType.DMA,
          pltpu.SemaphoreType.DMA,
      ],
  )
  def gather_kernel(idxt_hbm, table_hbm, out_hbm, idx_v, g0, g1, t0, t1,
                    isem, gsem0, gsem1, wsem0, wsem1):
    wid = lax.axis_index("s") * NUM_CORES + lax.axis_index("c")
    b0 = wid * WIN
    lane = lax.iota(jnp.int32, 16)
    rvec = [lane + 16 * k for k in range(NGRP)]

    pltpu.async_copy(idxt_hbm.at[:, pl.ds(b0, WIN)], idx_v, isem)
    pltpu.make_async_copy(idxt_hbm.at[:, pl.ds(b0, WIN)], idx_v, isem).wait()

    def start_gather(c, g, sem):
      pltpu.async_copy(
          table_hbm.at[idx_v.at[c // 2, pl.ds((c % 2) * CBLK, CBLK)]], g, sem)

    def wait_gather(c, g, sem):
      pltpu.make_async_copy(
          table_hbm.at[idx_v.at[c // 2, pl.ds((c % 2) * CBLK, CBLK)]], g,
          sem).wait()

    def retile(g, t):
      @plsc.parallel_loop(0, 8, unroll=1)
      def body(ds):
        ds_splat = jnp.full((16,), ds, jnp.int32)
        for dt in range(DIM // 8):
          col = ds_splat + (dt * 8)
          for k in range(NGRP):
            vals = plsc.load_gather(g, [rvec[k], col])
            t[dt, k >> 3, ds, pl.ds((k & 7) * 16, 16)] = vals

    def tile_col(c):
      return wid * (2 * NH) + (c % 2) * 2

    def start_write(c, t, sem):
      pltpu.async_copy(t, out_hbm.at[c // 2, :, pl.ds(tile_col(c), 2)], sem)

    def wait_write(c, t, sem):
      pltpu.make_async_copy(
          t, out_hbm.at[c // 2, :, pl.ds(tile_col(c), 2)], sem).wait()

    start_gather(0, g0, gsem0)
    start_gather(1, g1, gsem1)

    npair = NCHUNK // 2

    def body(i, carry):
      def half(c, g, t, gsem, wsem):
        wait_gather(c, g, gsem)

        @pl.when(i > 0)
        def _():
          wait_write(c - 2, t, wsem)

        retile(g, t)

        @pl.when(i < npair - 1)
        def _():
          start_gather(c + 2, g, gsem)

        start_write(c, t, wsem)

      half(2 * i, g0, t0, gsem0, wsem0)
      half(2 * i + 1, g1, t1, gsem1, wsem1)
      return carry

    lax.fori_loop(0, npair, body, 0)

    wait_write(NCHUNK - 2, t0, wsem0)
    wait_write(NCHUNK - 1, t1, wsem1)

  return gather_kernel


_gather = _make_kernel()


@jax.jit
def kernel(idx_input, table):
  idx_t = idx_input.T.astype(jnp.int32)
  out5 = _gather(idx_t, table)
  return out5.transpose(2, 4, 0, 1, 3).reshape(B, L, DIM)


# speedup vs baseline: 1.1372x; 1.1372x over previous
import functools

import jax
import jax.numpy as jnp
from jax import lax
from jax.experimental import pallas as pl
from jax.experimental.pallas import tpu as pltpu
from jax.experimental.pallas import tpu_sc as plsc

VOCAB_ROWS = 100002
DIM = 64
B = 16384
L = 50

NUM_CORES = 2
NUM_SUBCORES = 16
NW = NUM_CORES * NUM_SUBCORES
WIN = B // NW
CBLK = 256
NH = WIN // CBLK
NCHUNK = L * NH
NGRP = CBLK // 16


def _make_kernel():
  mesh = plsc.VectorSubcoreMesh(core_axis_name="c", subcore_axis_name="s")

  @functools.partial(
      pl.kernel,
      mesh=mesh,
      compiler_params=pltpu.CompilerParams(use_tc_tiling_on_sc=False,
                                           needs_layout_passes=False),
      out_type=jax.ShapeDtypeStruct((L, DIM // 8, B // 128, 8, 128),
                                    jnp.float32),
      scratch_types=[
          pltpu.VMEM((L, WIN), jnp.int32),
          pltpu.VMEM((CBLK, DIM), jnp.float32),
          pltpu.VMEM((CBLK, DIM), jnp.float32),
          pltpu.VMEM((DIM // 8, CBLK // 128, 8, 128), jnp.float32),
          pltpu.VMEM((DIM // 8, CBLK // 128, 8, 128), jnp.float32),
          pltpu.SemaphoreType.DMA,
          pltpu.SemaphoreType.DMA,
          pltpu.SemaphoreType.DMA,
          pltpu.SemaphoreType.DMA,
          pltpu.SemaphoreType.DMA,
      ],
  )
  def gather_kernel(idxt_hbm, table_hbm, out_hbm, idx_v, g0, g1, t0, t1,
                    isem, gsem0, gsem1, wsem0, wsem1):
    wid = lax.axis_index("s") * NUM_CORES + lax.axis_index("c")
    b0 = wid * WIN
    lane = lax.iota(jnp.int32, 16)
    rvec = [lane + 16 * k for k in range(NGRP)]

    pltpu.async_copy(idxt_hbm.at[:, pl.ds(b0, WIN)], idx_v, isem)
    pltpu.make_async_copy(idxt_hbm.at[:, pl.ds(b0, WIN)], idx_v, isem).wait()

    def start_gather(c, g, sem):
      pltpu.async_copy(
          table_hbm.at[idx_v.at[c // 2, pl.ds((c % 2) * CBLK, CBLK)]], g, sem)

    def wait_gather(c, g, sem):
      pltpu.make_async_copy(
          table_hbm.at[idx_v.at[c // 2, pl.ds((c % 2) * CBLK, CBLK)]], g,
          sem).wait()

    def retile(g, t):
      @plsc.parallel_loop(0, DIM, unroll=4)
      def body(d):
        dt = d >> 3
        ds = d & 7
        col = jnp.full((16,), d, jnp.int32)
        for k in range(NGRP):
          vals = plsc.load_gather(g, [rvec[k], col])
          t[dt, k >> 3, ds, pl.ds((k & 7) * 16, 16)] = vals

    def tile_col(c):
      return wid * (2 * NH) + (c % 2) * 2

    def start_write(c, t, sem):
      pltpu.async_copy(t, out_hbm.at[c // 2, :, pl.ds(tile_col(c), 2)], sem)

    def wait_write(c, t, sem):
      pltpu.make_async_copy(
          t, out_hbm.at[c // 2, :, pl.ds(tile_col(c), 2)], sem).wait()

    start_gather(0, g0, gsem0)
    start_gather(1, g1, gsem1)

    npair = NCHUNK // 2

    def body(i, carry):
      def half(c, g, t, gsem, wsem):
        wait_gather(c, g, gsem)

        @pl.when(i > 0)
        def _():
          wait_write(c - 2, t, wsem)

        retile(g, t)

        @pl.when(i < npair - 1)
        def _():
          start_gather(c + 2, g, gsem)

        start_write(c, t, wsem)

      half(2 * i, g0, t0, gsem0, wsem0)
      half(2 * i + 1, g1, t1, gsem1, wsem1)
      return carry

    lax.fori_loop(0, npair, body, 0)

    wait_write(NCHUNK - 2, t0, wsem0)
    wait_write(NCHUNK - 1, t1, wsem1)

  return gather_kernel


_gather = _make_kernel()


@jax.jit
def kernel(idx_input, table):
  idx_t = idx_input.T.astype(jnp.int32)
  out5 = _gather(idx_t, table)
  return out5.transpose(2, 4, 0, 1, 3).reshape(B, L, DIM)
